# baseline (speedup 1.0000x reference)
import functools

import jax
import jax.numpy as jnp
from jax import lax
from jax.experimental import pallas as pl
from jax.experimental.pallas import tpu as pltpu
from jax.experimental.pallas import tpu_sc as plsc

_INFO = plsc.get_sparse_core_info()
_NC = _INFO.num_cores
_NS = _INFO.num_subcores
_NW = _NC * _NS

_BT = 4096
_T = 200
_D = 64
_B = _BT * _T
_C = 128
_TR = _T // 8
_CB = _BT // _C
_UNITS = _TR * _CB
_UPW = _UNITS // _NW
_JPW = _UPW * 8
_PER_W = _JPW * _C


def _make_gather():
  mesh = plsc.VectorSubcoreMesh(core_axis_name="c", subcore_axis_name="s")

  @functools.partial(
      pl.kernel,
      mesh=mesh,
      out_type=jax.ShapeDtypeStruct((_T, 8, _CB, 8, _C), jnp.float32),
      scratch_types=[
          pltpu.VMEM((_PER_W,), jnp.int32),
          pltpu.VMEM((_PER_W,), jnp.int32),
          pltpu.VMEM((_C, 2 * _D), jnp.float32),
          pltpu.VMEM((_C, 2 * _D), jnp.float32),
          pltpu.VMEM((8, 8, _C), jnp.float32),
          pltpu.VMEM((8, 8, _C), jnp.float32),
          pltpu.SemaphoreType.DMA,
          pltpu.SemaphoreType.DMA,
          pltpu.SemaphoreType.DMA,
      ],
      compiler_params=pltpu.CompilerParams(
          use_tc_tiling_on_sc=True, needs_layout_passes=False
      ),
  )
  def gather_kernel(xg_hbm, xr_hbm, table_hbm, out_hbm, xg_v, xr_v, rows0,
                    rows1, tb0, tb1, gsem, ssem0, ssem1):
    w = lax.axis_index("s") * _NC + lax.axis_index("c")
    base = w * _PER_W
    pltpu.sync_copy(xg_hbm.at[pl.ds(base, _PER_W)], xg_v)
    pltpu.sync_copy(xr_hbm.at[pl.ds(base, _PER_W)], xr_v)

    rows = [rows0, rows1]
    tb = [tb0, tb1]
    ssem = [ssem0, ssem1]
    iv = lax.iota(jnp.int32, 16)
    gvecs = [iv + 16 * g for g in range(8)]

    def fire_gather(j, buf):
      pltpu.async_copy(
          table_hbm.at[xg_v.at[pl.ds(j * _C, _C)]], buf, gsem
      )

    def wait_gather(buf):
      pltpu.make_async_copy(
          table_hbm.at[xg_v.at[pl.ds(0, _C)]], buf, gsem
      ).wait()

    def out_slice(j):
      u = w * _UPW + j // 8
      t = (u // _CB) * 8 + j % 8
      return out_hbm.at[t, :, u % _CB, :, :]

    fire_gather(0, rows[0])

    @pl.loop(0, _JPW, step=2)
    def _unit(j0):
      for b in range(2):
        j = j0 + b

        @pl.when(j + 1 < _JPW)
        def _next():
          fire_gather(j + 1, rows[1 - b])

        pv = [plsc.load_gather(xr_v, [iv + (j * _C + 16 * m)])
              for m in range(8)]

        wait_gather(rows[b])

        @pl.when(j >= 2)
        def _drain():
          pltpu.make_async_copy(tb[b], out_slice(j - 2), ssem[b]).wait()

        @plsc.parallel_loop(0, 16, unroll=2)
        def _tp(k):
          evec = (iv + k) & 15
          rvec = evec >> 3
          e1vec = evec & 7
          for e0 in (0, 16, 32, 48):
            ev = evec + e0
            rv = rvec + (e0 >> 3)
            for m in range(8):
              v = plsc.load_gather(rows[b], [gvecs[m], ev + pv[m]])
              plsc.store_scatter(tb[b], [rv, e1vec, gvecs[m]], v)

        pltpu.async_copy(tb[b], out_slice(j), ssem[b])

    pltpu.make_async_copy(tb[0], out_slice(_JPW - 2), ssem[0]).wait()
    pltpu.make_async_copy(tb[1], out_slice(_JPW - 1), ssem[1]).wait()

  return gather_kernel


_gather = _make_gather()


@jax.jit
def kernel(X, tok_emb):
  xp = X.T.reshape(_TR, 8, _CB, _C).transpose(0, 2, 1, 3).reshape(-1)
  xp = xp.astype(jnp.int32)
  xg = xp >> 1
  xr = (xp & 1) << 6
  t2 = tok_emb.reshape(500000, 2 * _D)
  out5 = _gather(xg, xr, t2)
  return out5.transpose(2, 4, 0, 1, 3).reshape(_BT, _T, _D)


# speedup vs baseline: 1.0872x; 1.0872x over previous
import functools

import jax
import jax.numpy as jnp
from jax import lax
from jax.experimental import pallas as pl
from jax.experimental.pallas import tpu as pltpu
from jax.experimental.pallas import tpu_sc as plsc

_INFO = plsc.get_sparse_core_info()
_NC = _INFO.num_cores
_NS = _INFO.num_subcores
_NW = _NC * _NS

_BT = 4096
_T = 200
_D = 64
_B = _BT * _T
_C = 128
_TR = _T // 8
_CB = _BT // _C
_UNITS = _TR * _CB
_UPW = _UNITS // _NW
_JPW = _UPW * 8
_PER_W = _JPW * _C


def _make_gather():
  mesh = plsc.VectorSubcoreMesh(core_axis_name="c", subcore_axis_name="s")

  @functools.partial(
      pl.kernel,
      mesh=mesh,
      out_type=jax.ShapeDtypeStruct((_T, 8, _CB, 8, _C), jnp.float32),
      scratch_types=[
          pltpu.VMEM((_PER_W,), jnp.int32),
          pltpu.VMEM((_C, 2 * _D), jnp.float32),
          pltpu.VMEM((_C, 2 * _D), jnp.float32),
          pltpu.VMEM((8, 8, _C), jnp.float32),
          pltpu.VMEM((8, 8, _C), jnp.float32),
          pltpu.SemaphoreType.DMA,
          pltpu.SemaphoreType.DMA,
          pltpu.SemaphoreType.DMA,
      ],
      compiler_params=pltpu.CompilerParams(
          use_tc_tiling_on_sc=True, needs_layout_passes=False
      ),
  )
  def gather_kernel(idx_hbm, table_hbm, out_hbm, idx_v, rows0, rows1,
                    tb0, tb1, gsem, ssem0, ssem1):
    w = lax.axis_index("s") * _NC + lax.axis_index("c")
    base = w * _PER_W
    pltpu.sync_copy(idx_hbm.at[pl.ds(base, _PER_W)], idx_v)

    rows = [rows0, rows1]
    tb = [tb0, tb1]
    ssem = [ssem0, ssem1]
    iv = lax.iota(jnp.int32, 16)
    gvecs = [iv + 16 * g for g in range(8)]

    def fire_gather(j, buf):
      pltpu.async_copy(
          table_hbm.at[idx_v.at[pl.ds(j * _C, _C)]], buf, gsem
      )

    def wait_gather(buf):
      pltpu.make_async_copy(
          table_hbm.at[idx_v.at[pl.ds(0, _C)]], buf, gsem
      ).wait()

    def out_slice(j):
      u = w * _UPW + j // 8
      t = (u // _CB) * 8 + j % 8
      return out_hbm.at[t, :, u % _CB, :, :]

    fire_gather(0, rows[0])

    @pl.loop(0, _JPW, step=2)
    def _unit(j0):
      for b in range(2):
        j = j0 + b

        @pl.when(j + 1 < _JPW)
        def _next():
          fire_gather(j + 1, rows[1 - b])

        wait_gather(rows[b])

        @pl.when(j >= 2)
        def _drain():
          pltpu.make_async_copy(tb[b], out_slice(j - 2), ssem[b]).wait()

        @plsc.parallel_loop(0, 16, unroll=2)
        def _tp(k):
          evec = (iv + k) & 15
          rvec = evec >> 3
          e1vec = evec & 7
          for e0 in (0, 16, 32, 48):
            ev = evec + e0
            rv = rvec + (e0 >> 3)
            for m in range(8):
              v = plsc.load_gather(rows[b], [gvecs[m], ev])
              plsc.store_scatter(tb[b], [rv, e1vec, gvecs[m]], v)

        pltpu.async_copy(tb[b], out_slice(j), ssem[b])

    pltpu.make_async_copy(tb[0], out_slice(_JPW - 2), ssem[0]).wait()
    pltpu.make_async_copy(tb[1], out_slice(_JPW - 1), ssem[1]).wait()

  return gather_kernel


_gather = _make_gather()


@jax.jit
def kernel(X, tok_emb):
  xp = X.T.reshape(_TR, 8, _CB, _C).transpose(0, 2, 1, 3).reshape(-1)
  tpad = jnp.pad(tok_emb, ((0, 0), (0, _D)))
  out5 = _gather(xp.astype(jnp.int32), tpad)
  return out5.transpose(2, 4, 0, 1, 3).reshape(_BT, _T, _D)
